# baseline (speedup 1.0000x reference)
import functools

import jax
import jax.numpy as jnp
from jax import lax
from jax.experimental import pallas as pl
from jax.experimental.pallas import tpu as pltpu
from jax.experimental.pallas import tpu_sc as plsc

_NC = 2
_NS = 16
_NW = _NC * _NS


def _sc_gather(table, idx, batch):
    nw, n_chunks, lane = idx.shape
    n_idx = nw * n_chunks * lane
    d = table.shape[1]
    half = batch // 2

    mesh = plsc.VectorSubcoreMesh(core_axis_name="c", subcore_axis_name="s")

    @functools.partial(
        pl.kernel,
        mesh=mesh,
        out_type=jax.ShapeDtypeStruct((n_idx // 2, 2, d), jnp.float32),
        scratch_types=[
            pltpu.VMEM((n_chunks, lane), jnp.int32),
            pltpu.VMEM((2, lane, d), jnp.float32),
            pltpu.SemaphoreType.DMA,
            pltpu.SemaphoreType.DMA,
            pltpu.SemaphoreType.DMA,
            pltpu.SemaphoreType.DMA,
        ],
        compiler_params=pltpu.CompilerParams(use_tc_tiling_on_sc=False),
    )
    def gather_kernel(table_hbm, idx_hbm, out_hbm, idx_v, rows_v, gs0, gs1, os0, os1):
        wid = lax.axis_index("s") * _NC + lax.axis_index("c")
        per_w = n_chunks * lane
        base = wid * per_w
        pltpu.sync_copy(idx_hbm.at[wid], idx_v)
        gsems = (gs0, gs1)
        osems = (os0, os1)
        gathers = [None] * n_chunks
        outs = [None] * n_chunks
        gathers[0] = pltpu.async_copy(
            table_hbm.at[idx_v.at[0]], rows_v.at[0], gsems[0]
        )
        for j in range(n_chunks):
            b = j % 2
            nb = (j + 1) % 2
            if j + 1 < n_chunks:
                if outs[j - 1] is not None:
                    outs[j - 1].wait()
                gathers[j + 1] = pltpu.async_copy(
                    table_hbm.at[idx_v.at[j + 1]], rows_v.at[nb], gsems[nb]
                )
            gathers[j].wait()
            p0 = base + j * lane
            s0 = lax.rem(p0, batch)
            h = s0 // half
            r0 = pl.multiple_of(
                (p0 // batch) * half + lax.rem(s0, half), lane
            )
            outs[j] = pltpu.async_copy(
                rows_v.at[b], out_hbm.at[pl.ds(r0, lane), h], osems[b]
            )
        outs[n_chunks - 2].wait()
        outs[n_chunks - 1].wait()

    return gather_kernel(table, idx)


def _make_mlp(win, emb, bm, dcol):
    def mlp_body(*refs):
        x_refs = refs[:win]
        w0_ref, b0_ref, w1_ref, b1_ref, o_lo_ref, o_hi_ref = refs[win:]
        h1 = b0_ref.shape[0]
        bias = jnp.broadcast_to(b0_ref[...], (h1, bm)).astype(jnp.float32)
        acc_lo = bias
        acc_hi = bias
        for w in range(win):
            xw = x_refs[w]
            w0w = w0_ref[w * emb:(w + 1) * emb, :]
            acc_lo = acc_lo + lax.dot_general(
                w0w, xw[:, :emb],
                (((0,), (1,)), ((), ())),
                preferred_element_type=jnp.float32,
            )
            acc_hi = acc_hi + lax.dot_general(
                w0w, xw[:, dcol:dcol + emb],
                (((0,), (1,)), ((), ())),
                preferred_element_type=jnp.float32,
            )
        for acc, o_ref in ((acc_lo, o_lo_ref), (acc_hi, o_hi_ref)):
            h = jnp.tanh(acc)
            logits = lax.dot_general(
                w1_ref[...], h,
                (((0,), (0,)), ((), ())),
                preferred_element_type=jnp.float32,
            ) + b1_ref[...]
            shifted = logits - jnp.max(logits, axis=0, keepdims=True)
            lse = jnp.log(jnp.sum(jnp.exp(shifted), axis=0, keepdims=True))
            o_ref[...] = shifted - lse

    return mlp_body


def _mlp(x, batch, win, emb, w0, b0, w1, b1, bm):
    h1 = w0.shape[1]
    t = w1.shape[1]
    half = batch // 2
    d = x.shape[1]
    nb = half // bm
    x_specs = [
        pl.BlockSpec(
            (bm, d),
            functools.partial(lambda w, i: (w * nb + i, 0), w),
        )
        for w in range(win)
    ]
    out_sds = jax.ShapeDtypeStruct((t, half), jnp.float32)
    out_spec = pl.BlockSpec((t, bm), lambda i: (0, i))
    return pl.pallas_call(
        _make_mlp(win, emb, bm, d // 2),
        grid=(nb,),
        in_specs=x_specs + [
            pl.BlockSpec((win * emb, h1), lambda i: (0, 0)),
            pl.BlockSpec((h1, 1), lambda i: (0, 0)),
            pl.BlockSpec((h1, t), lambda i: (0, 0)),
            pl.BlockSpec((t, 1), lambda i: (0, 0)),
        ],
        out_specs=[out_spec, out_spec],
        out_shape=[out_sds, out_sds],
    )(*([x] * win), w0, b0.reshape(h1, 1), w1, b1.reshape(t, 1))


def _transpose_pad_body(et_ref, o_ref):
    xt = et_ref[...]
    emb, bn = xt.shape
    eye = (
        lax.broadcasted_iota(jnp.int32, (emb, emb), 0)
        == lax.broadcasted_iota(jnp.int32, (emb, emb), 1)
    ).astype(jnp.float32)
    t = lax.dot_general(
        xt, eye, (((0,), (0,)), ((), ())), preferred_element_type=jnp.float32
    )
    o_ref[...] = jnp.concatenate(
        [t, jnp.zeros((bn, o_ref.shape[1] - emb), jnp.float32)], axis=1
    )


def _transpose_pad(et, vocab_p, bn, d):
    emb = et.shape[0]
    return pl.pallas_call(
        _transpose_pad_body,
        grid=(vocab_p // bn,),
        in_specs=[pl.BlockSpec((emb, bn), lambda i: (0, i))],
        out_specs=pl.BlockSpec((bn, d), lambda i: (i, 0)),
        out_shape=jax.ShapeDtypeStruct((vocab_p, d), jnp.float32),
    )(et)


def kernel(v, E, W0, b0, W1, b1):
    batch, win = v.shape
    vocab, emb = E.shape
    half = batch // 2
    bn = 2048
    vocab_p = (vocab + bn - 1) // bn * bn
    e_pad = _transpose_pad(E.T, vocab_p, bn, 128)
    e64 = e_pad.reshape(vocab_p * 2, 64)
    idx = (v.T * 2).reshape(_NW, batch * win // (_NW * 128), 128)
    rows = _sc_gather(e64, idx, batch)
    x = rows.reshape(win * half, 128)
    lo, hi = _mlp(x, batch, win, emb, W0, b0, W1, b1, 2048)
    return jnp.concatenate([lo, hi], axis=1).T


# speedup vs baseline: 2.4271x; 2.4271x over previous
import functools

import jax
import jax.numpy as jnp
from jax import lax
from jax.experimental import pallas as pl
from jax.experimental.pallas import tpu as pltpu
from jax.experimental.pallas import tpu_sc as plsc

_NC = 2
_NS = 16
_NW = _NC * _NS


def _sc_gather(table, idx, batch):
    nw, n_chunks, lane = idx.shape
    n_idx = nw * n_chunks * lane
    d = table.shape[1]
    half = batch // 2

    mesh = plsc.VectorSubcoreMesh(core_axis_name="c", subcore_axis_name="s")

    @functools.partial(
        pl.kernel,
        mesh=mesh,
        out_type=jax.ShapeDtypeStruct((n_idx // 2, 2 * d), jnp.float32),
        scratch_types=[
            pltpu.VMEM((n_chunks, lane), jnp.int32),
            pltpu.VMEM((2, lane, d), jnp.float32),
            pltpu.SemaphoreType.DMA,
            pltpu.SemaphoreType.DMA,
            pltpu.SemaphoreType.DMA,
            pltpu.SemaphoreType.DMA,
        ],
        compiler_params=pltpu.CompilerParams(use_tc_tiling_on_sc=False),
    )
    def gather_kernel(table_hbm, idx_hbm, out_hbm, idx_v, rows_v, gs0, gs1, os0, os1):
        wid = lax.axis_index("s") * _NC + lax.axis_index("c")
        per_w = n_chunks * lane
        base = wid * per_w
        pltpu.sync_copy(idx_hbm.at[wid], idx_v)
        gsems = (gs0, gs1)
        osems = (os0, os1)
        gathers = [None] * n_chunks
        outs = [None] * n_chunks
        gathers[0] = pltpu.async_copy(
            table_hbm.at[idx_v.at[0]], rows_v.at[0], gsems[0]
        )
        for j in range(n_chunks):
            b = j % 2
            nb = (j + 1) % 2
            if j + 1 < n_chunks:
                if outs[j - 1] is not None:
                    outs[j - 1].wait()
                gathers[j + 1] = pltpu.async_copy(
                    table_hbm.at[idx_v.at[j + 1]], rows_v.at[nb], gsems[nb]
                )
            gathers[j].wait()
            p0 = base + j * lane
            s0 = lax.rem(p0, batch)
            h = s0 // half
            r0 = pl.multiple_of(
                (p0 // batch) * half + lax.rem(s0, half), lane
            )
            outs[j] = pltpu.async_copy(
                rows_v.at[b],
                out_hbm.at[pl.ds(r0, lane), pl.ds(h * d, d)],
                osems[b],
            )
        outs[n_chunks - 2].wait()
        outs[n_chunks - 1].wait()

    return gather_kernel(table, idx)


def _make_mlp(win, emb, bm, dcol):
    def mlp_body(*refs):
        x_refs = refs[:win]
        w0_ref, b0_ref, w1_ref, b1_ref, o_lo_ref, o_hi_ref = refs[win:]
        h1 = b0_ref.shape[0]
        bias = jnp.broadcast_to(b0_ref[...], (h1, bm)).astype(jnp.float32)
        acc_lo = bias
        acc_hi = bias
        for w in range(win):
            xw = x_refs[w]
            w0w = w0_ref[w * emb:(w + 1) * emb, :]
            acc_lo = acc_lo + lax.dot_general(
                w0w, xw[:, :emb],
                (((0,), (1,)), ((), ())),
                preferred_element_type=jnp.float32,
            )
            acc_hi = acc_hi + lax.dot_general(
                w0w, xw[:, dcol:dcol + emb],
                (((0,), (1,)), ((), ())),
                preferred_element_type=jnp.float32,
            )
        for acc, o_ref in ((acc_lo, o_lo_ref), (acc_hi, o_hi_ref)):
            h = jnp.tanh(acc)
            logits = lax.dot_general(
                w1_ref[...], h,
                (((0,), (0,)), ((), ())),
                preferred_element_type=jnp.float32,
            ) + b1_ref[...]
            shifted = logits - jnp.max(logits, axis=0, keepdims=True)
            lse = jnp.log(jnp.sum(jnp.exp(shifted), axis=0, keepdims=True))
            o_ref[...] = shifted - lse

    return mlp_body


def _mlp(x, batch, win, emb, w0, b0, w1, b1, bm):
    h1 = w0.shape[1]
    t = w1.shape[1]
    half = batch // 2
    d = x.shape[1]
    nb = half // bm
    x_specs = [
        pl.BlockSpec(
            (bm, d),
            functools.partial(lambda w, i: (w * nb + i, 0), w),
        )
        for w in range(win)
    ]
    out_sds = jax.ShapeDtypeStruct((t, half), jnp.float32)
    out_spec = pl.BlockSpec((t, bm), lambda i: (0, i))
    return pl.pallas_call(
        _make_mlp(win, emb, bm, d // 2),
        grid=(nb,),
        in_specs=x_specs + [
            pl.BlockSpec((win * emb, h1), lambda i: (0, 0)),
            pl.BlockSpec((h1, 1), lambda i: (0, 0)),
            pl.BlockSpec((h1, t), lambda i: (0, 0)),
            pl.BlockSpec((t, 1), lambda i: (0, 0)),
        ],
        out_specs=[out_spec, out_spec],
        out_shape=[out_sds, out_sds],
    )(*([x] * win), w0, b0.reshape(h1, 1), w1, b1.reshape(t, 1))


def _transpose_pad_body(et_ref, o_ref):
    xt = et_ref[...]
    emb, bn = xt.shape
    eye = (
        lax.broadcasted_iota(jnp.int32, (emb, emb), 0)
        == lax.broadcasted_iota(jnp.int32, (emb, emb), 1)
    ).astype(jnp.float32)
    t = lax.dot_general(
        xt, eye, (((0,), (0,)), ((), ())), preferred_element_type=jnp.float32
    )
    o_ref[...] = jnp.concatenate(
        [t, jnp.zeros((bn, o_ref.shape[1] - emb), jnp.float32)], axis=1
    )


def _transpose_pad(et, vocab_p, bn, d):
    emb = et.shape[0]
    return pl.pallas_call(
        _transpose_pad_body,
        grid=(vocab_p // bn,),
        in_specs=[pl.BlockSpec((emb, bn), lambda i: (0, i))],
        out_specs=pl.BlockSpec((bn, d), lambda i: (i, 0)),
        out_shape=jax.ShapeDtypeStruct((vocab_p, d), jnp.float32),
    )(et)


def kernel(v, E, W0, b0, W1, b1):
    batch, win = v.shape
    vocab, emb = E.shape
    half = batch // 2
    bn = 6272
    vocab_p = (vocab + bn - 1) // bn * bn
    ep = jnp.pad(E, ((0, vocab_p - vocab), (0, 0)))
    e_pad = _transpose_pad(ep.T, vocab_p, bn, 128)
    e64 = e_pad.reshape(vocab_p * 2, 64)
    idx = (v.T * 2).reshape(_NW, batch * win // (_NW * 128), 128)
    x = _sc_gather(e64, idx, batch)
    lo, hi = _mlp(x, batch, win, emb, W0, b0, W1, b1, 2048)
    return jnp.concatenate([lo, hi], axis=1).T
